# Initial kernel scaffold; baseline (speedup 1.0000x reference)
import jax
import jax.numpy as jnp
from jax.experimental import pallas as pl


def kernel(attr, state, edge_attr, params, edge_index):
    raise NotImplementedError("write your pallas kernel here")


# speedup vs baseline: 1.1362x; 1.1362x over previous
import functools

import jax
import jax.numpy as jnp
from jax.experimental import pallas as pl

NF = 64


def _node_pre_body(x8_ref, wr_ref, ws_ref, pe1_ref, pb1_ref, pe2_ref, pb2_ref,
                   a_ref, b_ref, pe_ref):
    x8 = x8_ref[...]
    a_ref[...] = jnp.dot(x8, wr_ref[...], preferred_element_type=jnp.float32)
    b_ref[...] = jnp.dot(x8, ws_ref[...], preferred_element_type=jnp.float32)
    t = jnp.maximum(jnp.dot(x8, pe1_ref[...], preferred_element_type=jnp.float32)
                    + pb1_ref[...], 0.0)
    pe_ref[...] = jnp.maximum(jnp.dot(t, pe2_ref[...], preferred_element_type=jnp.float32)
                              + pb2_ref[...], 0.0)


def _edge_dense_body(g_ref, ea_ref, we_ref, b1_ref, w2_ref, b2_ref, w3_ref,
                     b3_ref, wp1_ref, bp_ref, c_ref):
    h1 = jnp.maximum(g_ref[...] + ea_ref[...] * we_ref[...] + b1_ref[...], 0.0)
    h2 = jnp.maximum(jnp.dot(h1, w2_ref[...], preferred_element_type=jnp.float32)
                     + b2_ref[...], 0.0)
    re = jnp.maximum(jnp.dot(h2, w3_ref[...], preferred_element_type=jnp.float32)
                     + b3_ref[...], 0.0)
    c_ref[...] = jnp.dot(re, wp1_ref[...], preferred_element_type=jnp.float32) + bp_ref[...]


def _node_step_body(agg_ref, pe_ref, wq1_ref, wq2_ref, bq_ref, wp2_ref, wp3_ref,
                    u_ref, v_ref):
    eff = jnp.maximum(
        jnp.dot(pe_ref[...], wq1_ref[...], preferred_element_type=jnp.float32)
        + jnp.dot(agg_ref[...], wq2_ref[...], preferred_element_type=jnp.float32)
        + bq_ref[...], 0.0)
    u_ref[...] = jnp.dot(eff, wp2_ref[...], preferred_element_type=jnp.float32)
    v_ref[...] = jnp.dot(eff, wp3_ref[...], preferred_element_type=jnp.float32)


def _node_final_body(agg_ref, pe_ref, wq1_ref, wq2_ref, bq_ref,
                     pw1_ref, pb1_ref, pw2_ref, pb2_ref, pw3_ref, pb3_ref,
                     out_ref):
    eff = jnp.maximum(
        jnp.dot(pe_ref[...], wq1_ref[...], preferred_element_type=jnp.float32)
        + jnp.dot(agg_ref[...], wq2_ref[...], preferred_element_type=jnp.float32)
        + bq_ref[...], 0.0)
    t = jnp.maximum(jnp.dot(eff, pw1_ref[...], preferred_element_type=jnp.float32)
                    + pb1_ref[...], 0.0)
    t = jnp.maximum(jnp.dot(t, pw2_ref[...], preferred_element_type=jnp.float32)
                    + pb2_ref[...], 0.0)
    out_ref[...] = jnp.dot(t, pw3_ref[...], preferred_element_type=jnp.float32) + pb3_ref[...]


def _full_spec(shape):
    return pl.BlockSpec(shape, lambda i: tuple(0 for _ in shape))


def kernel(attr, state, edge_attr, params, edge_index):
    n = attr.shape[0]
    e = edge_attr.shape[0]
    recv = edge_index[0]
    send = edge_index[1]

    BN = 2048
    n_pad = ((n + BN - 1) // BN) * BN
    BE = 2048
    e_pad = ((e + BE - 1) // BE) * BE

    x8 = jnp.concatenate([attr, state[:, 3:6]], axis=1)
    x8 = jnp.pad(x8, ((0, n_pad - n), (0, 0)))

    W1, b1 = params["rel_enc"][0]
    W2, b2 = params["rel_enc"][1]
    W3, b3 = params["rel_enc"][2]
    Wr = jnp.concatenate([W1[0:5], W1[14:17]], axis=0)
    Ws = jnp.concatenate([W1[17:22], W1[31:34]], axis=0)
    we = W1[34:35]
    Pe1, pb1 = params["part_enc"][0]
    Pe2, pb2 = params["part_enc"][1]
    Pe1r = jnp.concatenate([Pe1[0:5], Pe1[14:17]], axis=0)
    Wp, bp = params["rel_prop"][0]
    Wp1, Wp2, Wp3 = Wp[0:64], Wp[64:128], Wp[128:192]
    Wq, bq = params["part_prop"][0]
    Wq1, Wq2 = Wq[0:64], Wq[64:128]
    (Pw1, Pb1), (Pw2, Pb2), (Pw3, Pb3) = params["predictor"]

    r2 = lambda v: v.reshape(1, -1)

    a_tab, b_tab, pe_tab = pl.pallas_call(
        _node_pre_body,
        grid=(n_pad // BN,),
        in_specs=[pl.BlockSpec((BN, 8), lambda i: (i, 0)),
                  _full_spec((8, NF)), _full_spec((8, NF)),
                  _full_spec((8, NF)), _full_spec((1, NF)),
                  _full_spec((NF, NF)), _full_spec((1, NF))],
        out_specs=[pl.BlockSpec((BN, NF), lambda i: (i, 0))] * 3,
        out_shape=[jax.ShapeDtypeStruct((n_pad, NF), jnp.float32)] * 3,
    )(x8, Wr, Ws, Pe1r, r2(pb1), Pe2, r2(pb2))

    g = jnp.take(a_tab, recv, axis=0) + jnp.take(b_tab, send, axis=0)
    g = jnp.pad(g, ((0, e_pad - e), (0, 0)))
    ea = jnp.pad(edge_attr, ((0, e_pad - e), (0, 0)))

    c = pl.pallas_call(
        _edge_dense_body,
        grid=(e_pad // BE,),
        in_specs=[pl.BlockSpec((BE, NF), lambda i: (i, 0)),
                  pl.BlockSpec((BE, 1), lambda i: (i, 0)),
                  _full_spec((1, NF)), _full_spec((1, NF)),
                  _full_spec((NF, NF)), _full_spec((1, NF)),
                  _full_spec((NF, NF)), _full_spec((1, NF)),
                  _full_spec((NF, NF)), _full_spec((1, NF))],
        out_specs=pl.BlockSpec((BE, NF), lambda i: (i, 0)),
        out_shape=jax.ShapeDtypeStruct((e_pad, NF), jnp.float32),
    )(g, ea, we, r2(b1), W2, r2(b2), W3, r2(b3), Wp1, r2(bp))
    c = c[:e]

    agg1 = jax.ops.segment_sum(jnp.maximum(c, 0.0), recv, num_segments=n)
    agg1 = jnp.pad(agg1, ((0, n_pad - n), (0, 0)))

    u_tab, v_tab = pl.pallas_call(
        _node_step_body,
        grid=(n_pad // BN,),
        in_specs=[pl.BlockSpec((BN, NF), lambda i: (i, 0)),
                  pl.BlockSpec((BN, NF), lambda i: (i, 0)),
                  _full_spec((NF, NF)), _full_spec((NF, NF)), _full_spec((1, NF)),
                  _full_spec((NF, NF)), _full_spec((NF, NF))],
        out_specs=[pl.BlockSpec((BN, NF), lambda i: (i, 0))] * 2,
        out_shape=[jax.ShapeDtypeStruct((n_pad, NF), jnp.float32)] * 2,
    )(agg1, pe_tab, Wq1, Wq2, r2(bq), Wp2, Wp3)

    rel2 = jnp.maximum(c + jnp.take(u_tab, recv, axis=0)
                       + jnp.take(v_tab, send, axis=0), 0.0)
    agg2 = jax.ops.segment_sum(rel2, recv, num_segments=n)
    agg2 = jnp.pad(agg2, ((0, n_pad - n), (0, 0)))

    pred = pl.pallas_call(
        _node_final_body,
        grid=(n_pad // BN,),
        in_specs=[pl.BlockSpec((BN, NF), lambda i: (i, 0)),
                  pl.BlockSpec((BN, NF), lambda i: (i, 0)),
                  _full_spec((NF, NF)), _full_spec((NF, NF)), _full_spec((1, NF)),
                  _full_spec((NF, NF)), _full_spec((1, NF)),
                  _full_spec((NF, NF)), _full_spec((1, NF)),
                  _full_spec((NF, 3)), _full_spec((1, 3))],
        out_specs=pl.BlockSpec((BN, 3), lambda i: (i, 0)),
        out_shape=jax.ShapeDtypeStruct((n_pad, 3), jnp.float32),
    )(agg2, pe_tab, Wq1, Wq2, r2(bq), Pw1, r2(Pb1), Pw2, r2(Pb2), Pw3, r2(Pb3))
    return pred[:n]


# speedup vs baseline: 2.6786x; 2.3575x over previous
import functools

import jax
import jax.numpy as jnp
from jax import lax
from jax.experimental import pallas as pl
from jax.experimental.pallas import tpu as pltpu
from jax.experimental.pallas import tpu_sc as plsc

NF = 64
HALF = 32

_NCORE = 2
_NSUB = 16
_NPAD = 51200
_EPAD = 802816
_ROWS = _EPAD // 128
_CH = 256
_CHR = _CH // 128
_BN = 2048
_BE = 2048


def _node_pre_body(x8_ref, wr_ref, ws_ref, pe1_ref, pb1_ref, pe2_ref, pb2_ref,
                   a_ref, b_ref, pe_ref):
    x8 = x8_ref[...]
    a_ref[...] = jnp.dot(x8, wr_ref[...], preferred_element_type=jnp.float32)
    b_ref[...] = jnp.dot(x8, ws_ref[...], preferred_element_type=jnp.float32)
    t = jnp.maximum(jnp.dot(x8, pe1_ref[...], preferred_element_type=jnp.float32)
                    + pb1_ref[...], 0.0)
    pe_ref[...] = jnp.maximum(jnp.dot(t, pe2_ref[...], preferred_element_type=jnp.float32)
                              + pb2_ref[...], 0.0)


def _edge_dense_body(g_ref, ea_ref, we_ref, b1_ref, w2_ref, b2_ref, w3_ref,
                     b3_ref, wp1_ref, bp_ref, c_ref):
    h = pl.program_id(0)
    h1 = jnp.maximum(g_ref[...] + ea_ref[...] * we_ref[...] + b1_ref[...], 0.0)
    h2 = jnp.maximum(jnp.dot(h1, w2_ref[...], preferred_element_type=jnp.float32)
                     + b2_ref[...], 0.0)
    del h
    re = jnp.maximum(jnp.dot(h2, w3_ref[...], preferred_element_type=jnp.float32)
                     + b3_ref[...], 0.0)
    c_ref[...] = (jnp.dot(re, wp1_ref[...], preferred_element_type=jnp.float32)
                  + bp_ref[0:1, :])


def _node_step_body(agg0_ref, agg1_ref, pe_ref, wq1_ref, wq2a_ref, wq2b_ref,
                    bq_ref, wp2_ref, wp3_ref, u_ref, v_ref):
    h = pl.program_id(0)
    eff = jnp.maximum(
        jnp.dot(pe_ref[...], wq1_ref[...], preferred_element_type=jnp.float32)
        + jnp.dot(agg0_ref[...], wq2a_ref[...], preferred_element_type=jnp.float32)
        + jnp.dot(agg1_ref[...], wq2b_ref[...], preferred_element_type=jnp.float32)
        + bq_ref[...], 0.0)
    del h
    u_ref[...] = jnp.dot(eff, wp2_ref[...], preferred_element_type=jnp.float32)
    v_ref[...] = jnp.dot(eff, wp3_ref[...], preferred_element_type=jnp.float32)


def _node_final_body(agg0_ref, agg1_ref, pe_ref, wq1_ref, wq2a_ref, wq2b_ref,
                     bq_ref, pw1_ref, pb1_ref, pw2_ref, pb2_ref, pw3_ref,
                     pb3_ref, out_ref):
    eff = jnp.maximum(
        jnp.dot(pe_ref[...], wq1_ref[...], preferred_element_type=jnp.float32)
        + jnp.dot(agg0_ref[...], wq2a_ref[...], preferred_element_type=jnp.float32)
        + jnp.dot(agg1_ref[...], wq2b_ref[...], preferred_element_type=jnp.float32)
        + bq_ref[...], 0.0)
    t = jnp.maximum(jnp.dot(eff, pw1_ref[...], preferred_element_type=jnp.float32)
                    + pb1_ref[...], 0.0)
    t = jnp.maximum(jnp.dot(t, pw2_ref[...], preferred_element_type=jnp.float32)
                    + pb2_ref[...], 0.0)
    out_ref[...] = jnp.dot(t, pw3_ref[...], preferred_element_type=jnp.float32) + pb3_ref[...]


def _full_spec(shape):
    return pl.BlockSpec(shape, lambda *a: tuple(0 for _ in shape))


def _sc_gather_body(a_hbm, b_hbm, ir_hbm, is_hbm, g_hbm,
                    ir_v, is_v, abuf, bbuf, sem):
    cid = lax.axis_index("c")
    sid = lax.axis_index("s")
    wid = sid * _NCORE + cid
    nch = _EPAD // (32 * _CH)
    row0 = wid * (nch * _CHR)
    e0 = wid * (nch * _CH)

    @pl.loop(0, nch)
    def _chunk(g):
        r = row0 + g * _CHR
        pltpu.sync_copy(ir_hbm.at[pl.ds(r, _CHR)], ir_v)
        pltpu.sync_copy(is_hbm.at[pl.ds(r, _CHR)], is_v)
        cps = []
        for j in range(_CHR):
            cps.append(pltpu.async_copy(a_hbm.at[ir_v.at[j]],
                                        abuf.at[pl.ds(j * 128, 128)], sem))
            cps.append(pltpu.async_copy(b_hbm.at[is_v.at[j]],
                                        bbuf.at[pl.ds(j * 128, 128)], sem))
        for cp in cps:
            cp.wait()

        @pl.loop(0, _CH, unroll=2)
        def _row(i):
            for j in range(NF // 16):
                sl = pl.ds(j * 16, 16)
                abuf[i, sl] = abuf[i, sl] + bbuf[i, sl]

        pltpu.sync_copy(abuf, g_hbm.at[pl.ds(e0 + g * _CH, _CH)])


def _sc_scatter1_body(c_hbm, iw_hbm, z_hbm, agg_hbm, iw_v, cbuf, acc, sem):
    del sem
    cid = lax.axis_index("c")
    sid = lax.axis_index("s")
    nrows_sub = _NPAD // _NSUB
    pltpu.sync_copy(z_hbm.at[pl.ds(sid * nrows_sub, nrows_sub)],
                    acc.at[pl.ds(sid * nrows_sub, nrows_sub)])
    plsc.subcore_barrier()
    nch = _EPAD // (_NSUB * _CH)
    row0 = sid * (nch * _CHR)
    e0 = cid * _EPAD + sid * (nch * _CH)

    @pl.loop(0, nch)
    def _chunk(g):
        pltpu.sync_copy(iw_hbm.at[pl.ds(row0 + g * _CHR, _CHR)], iw_v)
        pltpu.sync_copy(c_hbm.at[pl.ds(e0 + g * _CH, _CH)], cbuf)

        @pl.loop(0, _CH, unroll=2)
        def _row(i):
            for j in range(HALF // 16):
                sl = pl.ds(j * 16, 16)
                cbuf[i, sl] = jnp.maximum(cbuf[i, sl], 0.0)

        for j in range(_CHR):
            pltpu.sync_copy(cbuf.at[pl.ds(j * 128, 128)],
                            acc.at[iw_v.at[j]], add=True)

    plsc.subcore_barrier()
    pltpu.sync_copy(acc.at[pl.ds(sid * nrows_sub, nrows_sub)],
                    agg_hbm.at[pl.ds(cid * _NPAD + sid * nrows_sub, nrows_sub)])


def _sc_scatter2_body(c_hbm, u_hbm, v_hbm, ir_hbm, is_hbm, iw_hbm, z_hbm,
                      agg_hbm, ir_v, is_v, iw_v, cbuf, ubuf, vbuf, acc, sem):
    cid = lax.axis_index("c")
    sid = lax.axis_index("s")
    nrows_sub = _NPAD // _NSUB
    pltpu.sync_copy(z_hbm.at[pl.ds(sid * nrows_sub, nrows_sub)],
                    acc.at[pl.ds(sid * nrows_sub, nrows_sub)])
    plsc.subcore_barrier()
    nch = _EPAD // (_NSUB * _CH)
    row0 = sid * (nch * _CHR)
    orow0 = cid * _ROWS + row0
    e0 = cid * _EPAD + sid * (nch * _CH)

    @pl.loop(0, nch)
    def _chunk(g):
        r = row0 + g * _CHR
        orow = orow0 + g * _CHR
        pltpu.sync_copy(ir_hbm.at[pl.ds(orow, _CHR)], ir_v)
        pltpu.sync_copy(is_hbm.at[pl.ds(orow, _CHR)], is_v)
        pltpu.sync_copy(iw_hbm.at[pl.ds(r, _CHR)], iw_v)
        cps = []
        for j in range(_CHR):
            cps.append(pltpu.async_copy(u_hbm.at[ir_v.at[j]],
                                        ubuf.at[pl.ds(j * 128, 128)], sem))
            cps.append(pltpu.async_copy(v_hbm.at[is_v.at[j]],
                                        vbuf.at[pl.ds(j * 128, 128)], sem))
        pltpu.sync_copy(c_hbm.at[pl.ds(e0 + g * _CH, _CH)], cbuf)
        for cp in cps:
            cp.wait()

        @pl.loop(0, _CH, unroll=2)
        def _row(i):
            for j in range(HALF // 16):
                sl = pl.ds(j * 16, 16)
                cbuf[i, sl] = jnp.maximum(cbuf[i, sl] + ubuf[i, sl] + vbuf[i, sl], 0.0)

        for j in range(_CHR):
            pltpu.sync_copy(cbuf.at[pl.ds(j * 128, 128)],
                            acc.at[iw_v.at[j]], add=True)

    plsc.subcore_barrier()
    pltpu.sync_copy(acc.at[pl.ds(sid * nrows_sub, nrows_sub)],
                    agg_hbm.at[pl.ds(cid * _NPAD + sid * nrows_sub, nrows_sub)])


def kernel(attr, state, edge_attr, params, edge_index):
    n = attr.shape[0]
    e = edge_attr.shape[0]
    recv = edge_index[0]
    send = edge_index[1]

    x8 = jnp.concatenate([attr, state[:, 3:6]], axis=1)
    x8 = jnp.pad(x8, ((0, _NPAD - n), (0, 0)))

    W1, b1 = params["rel_enc"][0]
    W2, b2 = params["rel_enc"][1]
    W3, b3 = params["rel_enc"][2]
    Wr = jnp.concatenate([W1[0:5], W1[14:17]], axis=0)
    Ws = jnp.concatenate([W1[17:22], W1[31:34]], axis=0)
    we = W1[34:35]
    Pe1, pb1 = params["part_enc"][0]
    Pe2, pb2 = params["part_enc"][1]
    Pe1r = jnp.concatenate([Pe1[0:5], Pe1[14:17]], axis=0)
    Wp, bp = params["rel_prop"][0]
    Wp1, Wp2, Wp3 = Wp[0:64], Wp[64:128], Wp[128:192]
    Wq, bq = params["part_prop"][0]
    Wq1 = Wq[0:64]
    Wq2a, Wq2b = Wq[64:96], Wq[96:128]
    (Pw1, Pb1), (Pw2, Pb2), (Pw3, Pb3) = params["predictor"]

    r2 = lambda v: v.reshape(1, -1)

    recv_p = jnp.concatenate([recv, jnp.full((_EPAD - e,), n, jnp.int32)])
    send_p = jnp.concatenate([send, jnp.zeros((_EPAD - e,), jnp.int32)])
    ir2 = recv_p.reshape(_ROWS, 128)
    is2 = send_p.reshape(_ROWS, 128)
    iru = jnp.concatenate([recv_p, recv_p + _NPAD]).reshape(2 * _ROWS, 128)
    isu = jnp.concatenate([send_p, send_p + _NPAD]).reshape(2 * _ROWS, 128)
    zeros_half = jnp.zeros((_NPAD, HALF), jnp.float32)
    ea = jnp.pad(edge_attr, ((0, _EPAD - e), (0, 0)))

    mesh = plsc.VectorSubcoreMesh(core_axis_name="c", subcore_axis_name="s",
                                  num_cores=_NCORE, num_subcores=_NSUB)
    sc_params = pltpu.CompilerParams(use_tc_tiling_on_sc=False)

    a_tab, b_tab, pe_tab = pl.pallas_call(
        _node_pre_body,
        grid=(_NPAD // _BN,),
        in_specs=[pl.BlockSpec((_BN, 8), lambda i: (i, 0)),
                  _full_spec((8, NF)), _full_spec((8, NF)),
                  _full_spec((8, NF)), _full_spec((1, NF)),
                  _full_spec((NF, NF)), _full_spec((1, NF))],
        out_specs=[pl.BlockSpec((_BN, NF), lambda i: (i, 0))] * 3,
        out_shape=[jax.ShapeDtypeStruct((_NPAD, NF), jnp.float32)] * 3,
    )(x8, Wr, Ws, Pe1r, r2(pb1), Pe2, r2(pb2))

    sc_gather = functools.partial(
        pl.kernel, mesh=mesh, compiler_params=sc_params,
        out_type=jax.ShapeDtypeStruct((_EPAD, NF), jnp.float32),
        scratch_types=[pltpu.VMEM((_CHR, 128), jnp.int32),
                       pltpu.VMEM((_CHR, 128), jnp.int32),
                       pltpu.VMEM((_CH, NF), jnp.float32),
                       pltpu.VMEM((_CH, NF), jnp.float32),
                       pltpu.SemaphoreType.DMA],
    )(_sc_gather_body)
    g_edges = sc_gather(a_tab, b_tab, ir2, is2)

    c_s = pl.pallas_call(
        _edge_dense_body,
        grid=(2, _EPAD // _BE),
        in_specs=[pl.BlockSpec((_BE, NF), lambda h, i: (i, 0)),
                  pl.BlockSpec((_BE, 1), lambda h, i: (i, 0)),
                  _full_spec((1, NF)), _full_spec((1, NF)),
                  _full_spec((NF, NF)), _full_spec((1, NF)),
                  _full_spec((NF, NF)), _full_spec((1, NF)),
                  pl.BlockSpec((NF, HALF), lambda h, i: (h, 0)),
                  pl.BlockSpec((8, HALF), lambda h, i: (h, 0))],
        out_specs=pl.BlockSpec((_BE, HALF),
                               lambda h, i: (h * (_EPAD // _BE) + i, 0)),
        out_shape=jax.ShapeDtypeStruct((2 * _EPAD, HALF), jnp.float32),
    )(g_edges, ea, we, r2(b1), W2, r2(b2), W3, r2(b3),
      jnp.concatenate([Wp1[:, :HALF], Wp1[:, HALF:]], axis=0),
      jnp.zeros((16, HALF), jnp.float32).at[0].set(bp[:HALF]).at[8].set(bp[HALF:]))

    sc_scatter1 = functools.partial(
        pl.kernel, mesh=mesh, compiler_params=sc_params,
        out_type=jax.ShapeDtypeStruct((2 * _NPAD, HALF), jnp.float32),
        scratch_types=[pltpu.VMEM((_CHR, 128), jnp.int32),
                       pltpu.VMEM((_CH, HALF), jnp.float32),
                       pltpu.VMEM_SHARED((_NPAD, HALF), jnp.float32),
                       pltpu.SemaphoreType.DMA],
    )(_sc_scatter1_body)
    agg1_s = sc_scatter1(c_s, ir2, zeros_half)

    nb = _NPAD // _BN
    u_tab, v_tab = pl.pallas_call(
        _node_step_body,
        grid=(2, nb),
        in_specs=[pl.BlockSpec((_BN, HALF), lambda h, i: (i, 0)),
                  pl.BlockSpec((_BN, HALF), lambda h, i: (nb + i, 0)),
                  pl.BlockSpec((_BN, NF), lambda h, i: (i, 0)),
                  _full_spec((NF, NF)), _full_spec((HALF, NF)),
                  _full_spec((HALF, NF)), _full_spec((1, NF)),
                  pl.BlockSpec((NF, HALF), lambda h, i: (h, 0)),
                  pl.BlockSpec((NF, HALF), lambda h, i: (h, 0))],
        out_specs=[pl.BlockSpec((_BN, HALF), lambda h, i: (h * nb + i, 0))] * 2,
        out_shape=[jax.ShapeDtypeStruct((2 * _NPAD, HALF), jnp.float32)] * 2,
    )(agg1_s, agg1_s, pe_tab, Wq1, Wq2a, Wq2b, r2(bq),
      jnp.concatenate([Wp2[:, :HALF], Wp2[:, HALF:]], axis=0),
      jnp.concatenate([Wp3[:, :HALF], Wp3[:, HALF:]], axis=0))

    sc_scatter2 = functools.partial(
        pl.kernel, mesh=mesh, compiler_params=sc_params,
        out_type=jax.ShapeDtypeStruct((2 * _NPAD, HALF), jnp.float32),
        scratch_types=[pltpu.VMEM((_CHR, 128), jnp.int32),
                       pltpu.VMEM((_CHR, 128), jnp.int32),
                       pltpu.VMEM((_CHR, 128), jnp.int32),
                       pltpu.VMEM((_CH, HALF), jnp.float32),
                       pltpu.VMEM((_CH, HALF), jnp.float32),
                       pltpu.VMEM((_CH, HALF), jnp.float32),
                       pltpu.VMEM_SHARED((_NPAD, HALF), jnp.float32),
                       pltpu.SemaphoreType.DMA],
    )(_sc_scatter2_body)
    agg2_s = sc_scatter2(c_s, u_tab, v_tab, iru, isu, ir2, zeros_half)

    pred = pl.pallas_call(
        _node_final_body,
        grid=(nb,),
        in_specs=[pl.BlockSpec((_BN, HALF), lambda i: (i, 0)),
                  pl.BlockSpec((_BN, HALF), lambda i: (nb + i, 0)),
                  pl.BlockSpec((_BN, NF), lambda i: (i, 0)),
                  _full_spec((NF, NF)), _full_spec((HALF, NF)),
                  _full_spec((HALF, NF)), _full_spec((1, NF)),
                  _full_spec((NF, NF)), _full_spec((1, NF)),
                  _full_spec((NF, NF)), _full_spec((1, NF)),
                  _full_spec((NF, 3)), _full_spec((1, 3))],
        out_specs=pl.BlockSpec((_BN, 3), lambda i: (i, 0)),
        out_shape=jax.ShapeDtypeStruct((_NPAD, 3), jnp.float32),
    )(agg2_s, agg2_s, pe_tab, Wq1, Wq2a, Wq2b, r2(bq),
      Pw1, r2(Pb1), Pw2, r2(Pb2), Pw3, r2(Pb3))
    return pred[:n]
